# baseline (speedup 1.0000x reference)
import functools

import jax
import jax.numpy as jnp
from jax import lax
from jax.experimental import pallas as pl
from jax.experimental.pallas import tpu as pltpu
from jax.experimental.pallas import tpu_sc as plsc

N = 10000
D = 128
NP = 4
DQ = D // NP
E = 320000
NC = 2
NS = 16
CHUNK = 80
NBUF = 5
EPT = E // NS
NCHUNK = EPT // CHUNK
ROWS_PT = 624
TAIL0 = NS * ROWS_PT
TAIL = N - TAIL0
RB = 1000


def _tc_body(xu_ref, xi_ref, wn_ref, ws_ref, b_ref, *out_refs):
    g = pl.program_id(0)
    x = jnp.where(g == 0, xu_ref[...], xi_ref[...])
    y = jnp.dot(x, wn_ref[0], preferred_element_type=jnp.float32)
    base = jnp.dot(x, ws_ref[0], preferred_element_type=jnp.float32) + b_ref[0]
    for q in range(NP):
        out_refs[q][...] = y[:, q * DQ:(q + 1) * DQ]
        out_refs[NP + q][...] = base[:, q * DQ:(q + 1) * DQ]


def _tc_dense(x_user, x_item, wn_all, ws_all, b_all):
    quarter = jax.ShapeDtypeStruct((2 * N, DQ), jnp.float32)
    nrb = N // RB
    return pl.pallas_call(
        _tc_body,
        grid=(2, nrb),
        in_specs=[
            pl.BlockSpec((RB, D), lambda g, r: ((1 - g) * r, 0)),
            pl.BlockSpec((RB, D), lambda g, r: (g * r, 0)),
            pl.BlockSpec((1, D, D), lambda g, r: (g, 0, 0)),
            pl.BlockSpec((1, D, D), lambda g, r: (g, 0, 0)),
            pl.BlockSpec((1, 1, D), lambda g, r: (g, 0, 0)),
        ],
        out_specs=(
            [pl.BlockSpec((RB, DQ), lambda g, r: (g * nrb + r, 0))] * NP
            + [pl.BlockSpec((RB, DQ), lambda g, r: ((1 - g) * nrb + r, 0))] * NP
        ),
        out_shape=[quarter] * (2 * NP),
    )(x_user, x_item, wn_all, ws_all, b_all)


def _sc_body(y0_hbm, y1_hbm, y2_hbm, y3_hbm, b0_hbm, b1_hbm, b2_hbm, b3_hbm,
             src_hbm, dst_hbm, ew_hbm,
             o0_hbm, o1_hbm, o2_hbm, o3_hbm,
             idx_src, idx_dst, ew_v, rows_v, acc, ytab,
             semg0, semg1, semg2, semg3, semg4,
             sems0, sems1, sems2, sems3, sems4):
    c = lax.axis_index("c")
    s = lax.axis_index("s")
    w = c * NS + s
    row0 = c * N + s * ROWS_PT
    semg = (semg0, semg1, semg2, semg3, semg4)
    sems = (sems0, sems1, sems2, sems3, sems4)

    pltpu.sync_copy(src_hbm.at[w], idx_src)
    pltpu.sync_copy(dst_hbm.at[w], idx_dst)
    pltpu.sync_copy(ew_hbm.at[w], ew_v)

    for y_hbm, b_hbm, o_hbm in ((y0_hbm, b0_hbm, o0_hbm),
                                (y1_hbm, b1_hbm, o1_hbm),
                                (y2_hbm, b2_hbm, o2_hbm),
                                (y3_hbm, b3_hbm, o3_hbm)):
        pltpu.sync_copy(b_hbm.at[pl.ds(row0, ROWS_PT)],
                        acc.at[pl.ds(s * ROWS_PT, ROWS_PT)])
        pltpu.sync_copy(y_hbm.at[pl.ds(row0, ROWS_PT)],
                        ytab.at[pl.ds(s * ROWS_PT, ROWS_PT)])

        @pl.when(s == NS - 1)
        def _init_tail():
            pltpu.sync_copy(b_hbm.at[pl.ds(c * N + TAIL0, TAIL)],
                            acc.at[pl.ds(TAIL0, TAIL)])
            pltpu.sync_copy(y_hbm.at[pl.ds(c * N + TAIL0, TAIL)],
                            ytab.at[pl.ds(TAIL0, TAIL)])

        plsc.subcore_barrier()

        def g_issue(j, b_):
            pltpu.async_copy(ytab.at[idx_src.at[j]],
                             rows_v.at[pl.ds(b_ * CHUNK, CHUNK)], semg[b_])

        def g_wait(b_):
            pltpu.make_async_copy(y_hbm.at[pl.ds(0, CHUNK)],
                                  rows_v.at[pl.ds(b_ * CHUNK, CHUNK)],
                                  semg[b_]).wait()

        def s_issue(j, b_):
            pltpu.async_copy(rows_v.at[pl.ds(b_ * CHUNK, CHUNK)],
                             acc.at[idx_dst.at[j]], sems[b_], add=True)

        def s_wait(b_):
            pltpu.make_async_copy(y_hbm.at[pl.ds(0, CHUNK)],
                                  rows_v.at[pl.ds(b_ * CHUNK, CHUNK)],
                                  sems[b_]).wait()

        for b_ in range(NBUF):
            g_issue(b_, b_)

        def chunk_step(j, b):
            bb = b * CHUNK
            for b_ in range(NBUF):
                @pl.when(b == b_)
                def _gw(b_=b_):
                    g_wait(b_)
            for gi in range(CHUNK // 16):
                w16 = ew_v[pl.ds(j * CHUNK + gi * 16, 16)]
                for l in range(16):
                    wspl = w16.at[jnp.full((16,), l, jnp.int32)].get(
                        mode="promise_in_bounds")
                    e = gi * 16 + l
                    for d in range(DQ // 16):
                        sl = pl.ds(d * 16, 16)
                        rows_v[bb + e, sl] = rows_v[bb + e, sl] * wspl
            reissue = jnp.logical_and(j >= 1, j <= NCHUNK - NBUF)
            bp = jnp.where(b == 0, NBUF - 1, b - 1)
            for b_ in range(NBUF):
                @pl.when(b == b_)
                def _si(b_=b_):
                    s_issue(j, b_)
                @pl.when(jnp.logical_and(reissue, bp == b_))
                def _ri(b_=b_):
                    s_wait(b_)
                    g_issue(j - 1 + NBUF, b_)
            return jnp.where(b == NBUF - 1, 0, b + 1)

        lax.fori_loop(0, NCHUNK, chunk_step, 0)
        for b_ in range(NBUF):
            s_wait(b_)
        plsc.subcore_barrier()

        pltpu.sync_copy(acc.at[pl.ds(s * ROWS_PT, ROWS_PT)],
                        o_hbm.at[pl.ds(row0, ROWS_PT)])

        @pl.when(s == NS - 1)
        def _write_tail():
            pltpu.sync_copy(acc.at[pl.ds(TAIL0, TAIL)],
                            o_hbm.at[pl.ds(c * N + TAIL0, TAIL)])

        plsc.subcore_barrier()


_sc_agg = functools.partial(
    pl.kernel,
    out_type=[jax.ShapeDtypeStruct((2 * N, DQ), jnp.float32)] * NP,
    mesh=plsc.VectorSubcoreMesh(
        core_axis_name="c", subcore_axis_name="s", num_cores=NC,
        num_subcores=NS),
    compiler_params=pltpu.CompilerParams(use_tc_tiling_on_sc=False),
    scratch_types=[
        pltpu.VMEM((NCHUNK, CHUNK), jnp.int32),
        pltpu.VMEM((NCHUNK, CHUNK), jnp.int32),
        pltpu.VMEM((EPT,), jnp.float32),
        pltpu.VMEM((NBUF * CHUNK, DQ), jnp.float32),
        pltpu.VMEM_SHARED((N, DQ), jnp.float32),
        pltpu.VMEM_SHARED((N, DQ), jnp.float32),
        pltpu.SemaphoreType.DMA,
        pltpu.SemaphoreType.DMA,
        pltpu.SemaphoreType.DMA,
        pltpu.SemaphoreType.DMA,
        pltpu.SemaphoreType.DMA,
        pltpu.SemaphoreType.DMA,
        pltpu.SemaphoreType.DMA,
        pltpu.SemaphoreType.DMA,
        pltpu.SemaphoreType.DMA,
        pltpu.SemaphoreType.DMA,
    ],
)(_sc_body)


def kernel(x_user, x_item, edge_index_ui, edge_index_iu, ew_ui, ew_iu,
           W_nbr_ui, W_self_ui, b_ui, W_nbr_iu, W_self_iu, b_iu):
    wn_all = jnp.stack([W_nbr_ui, W_nbr_iu])
    ws_all = jnp.stack([W_self_iu, W_self_ui])
    b_all = jnp.stack([b_iu, b_ui])[:, None, :]
    outs = _tc_dense(x_user, x_item, wn_all, ws_all, b_all)
    ys = outs[:NP]
    bs = outs[NP:]

    src_ui = edge_index_ui[0].astype(jnp.int32).reshape(NS, EPT)
    dst_ui = edge_index_ui[1].astype(jnp.int32).reshape(NS, EPT)
    src_iu = edge_index_iu[0].astype(jnp.int32).reshape(NS, EPT)
    dst_iu = edge_index_iu[1].astype(jnp.int32).reshape(NS, EPT)
    src3 = jnp.concatenate([src_ui, src_iu]).reshape(2 * NS, NCHUNK, CHUNK)
    dst3 = jnp.concatenate([dst_ui, dst_iu]).reshape(2 * NS, NCHUNK, CHUNK)
    ew3 = jnp.concatenate([ew_ui.reshape(NS, EPT), ew_iu.reshape(NS, EPT)])

    os_ = _sc_agg(*ys, *bs, src3, dst3, ew3)
    out_cat = jnp.concatenate(os_, axis=1)
    out_item = out_cat[:N]
    out_user = out_cat[N:]
    return (out_user, out_item)


# speedup vs baseline: 1.3505x; 1.3505x over previous
import functools

import jax
import jax.numpy as jnp
from jax import lax
from jax.experimental import pallas as pl
from jax.experimental.pallas import tpu as pltpu
from jax.experimental.pallas import tpu_sc as plsc

N = 10000
D = 128
DH = D // 2
E = 320000
NC = 2
NS = 16
CHUNK = 80
NBUF = 5
EPT = E // NS
NCHUNK = 250
EPTP = NCHUNK * CHUNK
ROWS_PT = 624
TAIL0 = NS * ROWS_PT
TAIL = N - TAIL0
RB = 1000


def _tc_body(xu_ref, xi_ref, wn_ref, ws_ref, b_ref,
             y0_ref, y1_ref, b0_ref, b1_ref):
    g = pl.program_id(0)
    x = jnp.where(g == 0, xu_ref[...], xi_ref[...])
    y = jnp.dot(x, wn_ref[0], preferred_element_type=jnp.float32)
    y0_ref[...] = y[:, :DH]
    y1_ref[...] = y[:, DH:]
    base = jnp.dot(x, ws_ref[0], preferred_element_type=jnp.float32) + b_ref[0]
    b0_ref[...] = base[:, :DH]
    b1_ref[...] = base[:, DH:]


def _tc_dense(x_user, x_item, wn_all, ws_all, b_all):
    half = jax.ShapeDtypeStruct((2 * N, DH), jnp.float32)
    nrb = N // RB
    return pl.pallas_call(
        _tc_body,
        grid=(2, nrb),
        in_specs=[
            pl.BlockSpec((RB, D), lambda g, r: ((1 - g) * r, 0)),
            pl.BlockSpec((RB, D), lambda g, r: (g * r, 0)),
            pl.BlockSpec((1, D, D), lambda g, r: (g, 0, 0)),
            pl.BlockSpec((1, D, D), lambda g, r: (g, 0, 0)),
            pl.BlockSpec((1, 1, D), lambda g, r: (g, 0, 0)),
        ],
        out_specs=[
            pl.BlockSpec((RB, DH), lambda g, r: (g * nrb + r, 0)),
            pl.BlockSpec((RB, DH), lambda g, r: (g * nrb + r, 0)),
            pl.BlockSpec((RB, DH), lambda g, r: ((1 - g) * nrb + r, 0)),
            pl.BlockSpec((RB, DH), lambda g, r: ((1 - g) * nrb + r, 0)),
        ],
        out_shape=[half, half, half, half],
    )(x_user, x_item, wn_all, ws_all, b_all)


def _sc_body(y0_hbm, y1_hbm, b0_hbm, b1_hbm, src_hbm, dst_hbm, ew_hbm,
             o0_hbm, o1_hbm, idx_src, idx_dst, ew_v, rows_v, acc,
             semg0, semg1, semg2, semg3, semg4,
             sems0, sems1, sems2, sems3, sems4):
    c = lax.axis_index("c")
    s = lax.axis_index("s")
    w = c * NS + s
    row0 = c * N + s * ROWS_PT
    semg = (semg0, semg1, semg2, semg3, semg4)
    sems = (sems0, sems1, sems2, sems3, sems4)

    pltpu.sync_copy(src_hbm.at[w], idx_src)
    pltpu.sync_copy(dst_hbm.at[w], idx_dst)
    pltpu.sync_copy(ew_hbm.at[w], ew_v)

    for y_hbm, b_hbm, o_hbm in ((y0_hbm, b0_hbm, o0_hbm),
                                (y1_hbm, b1_hbm, o1_hbm)):
        pltpu.sync_copy(b_hbm.at[pl.ds(row0, ROWS_PT)],
                        acc.at[pl.ds(s * ROWS_PT, ROWS_PT)])

        @pl.when(s == NS - 1)
        def _init_tail():
            pltpu.sync_copy(b_hbm.at[pl.ds(c * N + TAIL0, TAIL)],
                            acc.at[pl.ds(TAIL0, TAIL)])

        plsc.subcore_barrier()

        def g_issue(j, b_):
            pltpu.async_copy(y_hbm.at[idx_src.at[j]],
                             rows_v.at[pl.ds(b_ * CHUNK, CHUNK)], semg[b_])

        def g_wait(j, b_):
            pltpu.make_async_copy(y_hbm.at[idx_src.at[j]],
                                  rows_v.at[pl.ds(b_ * CHUNK, CHUNK)],
                                  semg[b_]).wait()

        def s_issue(j, b_):
            pltpu.async_copy(rows_v.at[pl.ds(b_ * CHUNK, CHUNK)],
                             acc.at[idx_dst.at[j]], sems[b_], add=True)

        def s_wait(b_):
            pltpu.make_async_copy(y_hbm.at[pl.ds(0, CHUNK)],
                                  rows_v.at[pl.ds(b_ * CHUNK, CHUNK)],
                                  sems[b_]).wait()

        for b_ in range(NBUF):
            g_issue(b_, b_)

        def chunk_step(j, b):
            bb = b * CHUNK
            for b_ in range(NBUF):
                @pl.when(b == b_)
                def _gw(b_=b_):
                    g_wait(j, b_)
            for gi in range(CHUNK // 16):
                w16 = ew_v[pl.ds(j * CHUNK + gi * 16, 16)]
                for l in range(16):
                    wspl = w16.at[jnp.full((16,), l, jnp.int32)].get(
                        mode="promise_in_bounds")
                    e = gi * 16 + l
                    for d in range(DH // 16):
                        sl = pl.ds(d * 16, 16)
                        rows_v[bb + e, sl] = rows_v[bb + e, sl] * wspl
            reissue = jnp.logical_and(j >= 1, j <= NCHUNK - NBUF)
            bp = jnp.where(b == 0, NBUF - 1, b - 1)
            for b_ in range(NBUF):
                @pl.when(b == b_)
                def _si(b_=b_):
                    s_issue(j, b_)
                @pl.when(jnp.logical_and(reissue, bp == b_))
                def _ri(b_=b_):
                    s_wait(b_)
                    g_issue(j - 1 + NBUF, b_)
            return jnp.where(b == NBUF - 1, 0, b + 1)

        lax.fori_loop(0, NCHUNK, chunk_step, 0)
        for b_ in range(NBUF):
            s_wait(b_)
        plsc.subcore_barrier()

        pltpu.sync_copy(acc.at[pl.ds(s * ROWS_PT, ROWS_PT)],
                        o_hbm.at[pl.ds(row0, ROWS_PT)])

        @pl.when(s == NS - 1)
        def _write_tail():
            pltpu.sync_copy(acc.at[pl.ds(TAIL0, TAIL)],
                            o_hbm.at[pl.ds(c * N + TAIL0, TAIL)])

        plsc.subcore_barrier()


_sc_agg = functools.partial(
    pl.kernel,
    out_type=[jax.ShapeDtypeStruct((2 * N, DH), jnp.float32),
              jax.ShapeDtypeStruct((2 * N, DH), jnp.float32)],
    mesh=plsc.VectorSubcoreMesh(
        core_axis_name="c", subcore_axis_name="s", num_cores=NC,
        num_subcores=NS),
    compiler_params=pltpu.CompilerParams(use_tc_tiling_on_sc=False),
    scratch_types=[
        pltpu.VMEM((NCHUNK, CHUNK), jnp.int32),
        pltpu.VMEM((NCHUNK, CHUNK), jnp.int32),
        pltpu.VMEM((EPTP,), jnp.float32),
        pltpu.VMEM((NBUF * CHUNK, DH), jnp.float32),
        pltpu.VMEM_SHARED((N, DH), jnp.float32),
        pltpu.SemaphoreType.DMA,
        pltpu.SemaphoreType.DMA,
        pltpu.SemaphoreType.DMA,
        pltpu.SemaphoreType.DMA,
        pltpu.SemaphoreType.DMA,
        pltpu.SemaphoreType.DMA,
        pltpu.SemaphoreType.DMA,
        pltpu.SemaphoreType.DMA,
        pltpu.SemaphoreType.DMA,
        pltpu.SemaphoreType.DMA,
    ],
)(_sc_body)


def kernel(x_user, x_item, edge_index_ui, edge_index_iu, ew_ui, ew_iu,
           W_nbr_ui, W_self_ui, b_ui, W_nbr_iu, W_self_iu, b_iu):
    wn_all = jnp.stack([W_nbr_ui, W_nbr_iu])
    ws_all = jnp.stack([W_self_iu, W_self_ui])
    b_all = jnp.stack([b_iu, b_ui])[:, None, :]
    y0, y1, base0, base1 = _tc_dense(x_user, x_item, wn_all, ws_all, b_all)

    src_ui = edge_index_ui[0].astype(jnp.int32).reshape(NS, EPT)
    dst_ui = edge_index_ui[1].astype(jnp.int32).reshape(NS, EPT)
    src_iu = (edge_index_iu[0].astype(jnp.int32) + N).reshape(NS, EPT)
    dst_iu = edge_index_iu[1].astype(jnp.int32).reshape(NS, EPT)
    src3 = jnp.concatenate([src_ui, src_iu]).reshape(2 * NS, NCHUNK, CHUNK)
    dst3 = jnp.concatenate([dst_ui, dst_iu]).reshape(2 * NS, NCHUNK, CHUNK)
    ew3 = jnp.concatenate([ew_ui.reshape(NS, EPT), ew_iu.reshape(NS, EPT)])

    out0, out1 = _sc_agg(y0, y1, base0, base1, src3, dst3, ew3)
    out_cat = jnp.concatenate([out0, out1], axis=1)
    out_item = out_cat[:N]
    out_user = out_cat[N:]
    return (out_user, out_item)
